# Initial kernel scaffold; baseline (speedup 1.0000x reference)
import jax
import jax.numpy as jnp
from jax.experimental import pallas as pl


def kernel(pred0, pred1, pred2, targets):
    raise NotImplementedError("write your pallas kernel here")


# speedup vs baseline: 143.3213x; 143.3213x over previous
import functools

import numpy as np
import jax
import jax.numpy as jnp
from jax.experimental import pallas as pl

_ANCH = np.array([
    [[3.625, 2.8125], [4.875, 6.1875], [11.65625, 10.1875]],
    [[1.875, 3.8125], [3.875, 2.8125], [3.6875, 7.4375]],
    [[1.25, 1.625], [2.0, 3.75], [4.125, 2.875]],
], dtype=np.float32)
_BASE9 = np.array(
    [[10, 13], [16, 30], [33, 23], [30, 61], [62, 45], [59, 119],
     [116, 90], [156, 198], [373, 326]], dtype=np.float32)
_STRIDE = (32.0, 16.0, 8.0)
_MG = (2, 1, 0)
_IGN = 0.7
_N = 50


def _loss_body(pred_ref, tgt_ref, rwh_ref, out_ref, *, ny, nx, lyr):
    N = _N
    C = 3 * ny * nx
    fnx = float(nx)
    fny = float(ny)
    b = pl.program_id(0)
    t5 = tgt_ref[0]

    nt = jnp.sum((jnp.sum(t5, axis=1) > 0).astype(jnp.float32))
    tio = jax.lax.broadcasted_iota(jnp.int32, (N, 1), 0).astype(jnp.float32)
    validf = (tio < nt).astype(jnp.float32)
    l0 = t5[:, 0:1]
    lx = t5[:, 1:2] * fnx
    ly = t5[:, 2:3] * fny
    lw = t5[:, 3:4] * fnx
    lh = t5[:, 4:5] * fny

    rw = rwh_ref[0:1, :]
    rh = rwh_ref[1:2, :]
    bw = jnp.minimum(lw, rw)
    bh = jnp.minimum(lh, rh)
    en9 = ((bw > 0.0) & (bh > 0.0)).astype(jnp.float32)
    inter9 = bw * bh * en9
    iou9 = inter9 / (lw * lh + rw * rh - inter9)
    rowmax = jnp.max(iou9, axis=1, keepdims=True)
    i9 = jax.lax.broadcasted_iota(jnp.int32, (N, 9), 1)
    best = jnp.min(jnp.where(iou9 == rowmax, i9, 9), axis=1, keepdims=True)
    m = (best // 3) == _MG[lyr]
    best3 = (best - 3 * (best // 3)).astype(jnp.float32)
    okf = ((tio < nt) & m).astype(jnp.float32)
    any_m = jnp.max(okf)

    i_f = jnp.floor(lx)
    j_f = jnp.floor(ly)
    cell = (best3 * fny + j_f) * fnx + i_f

    ir = jax.lax.broadcasted_iota(jnp.int32, (N, N), 0)
    ic = jax.lax.broadcasted_iota(jnp.int32, (N, N), 1)
    eyef = (ir == ic).astype(jnp.float32)
    cell_r = jnp.sum(eyef * cell, axis=0, keepdims=True)
    ok_r = jnp.sum(eyef * okf, axis=0, keepdims=True)
    later_same = ((cell == cell_r) & (ok_r > 0.0) & (ir < ic)).astype(jnp.float32)
    ow = jnp.max(later_same, axis=1, keepdims=True)
    fin = okf * (1.0 - ow)

    sc = jnp.sqrt(2.0 - lw * lh / (fnx * fny))
    tex = lx - i_f
    tey = ly - j_f
    a = _ANCH[lyr]
    is1 = best3 == 1.0
    is2 = best3 == 2.0
    aw = jnp.where(is2, a[2, 0], jnp.where(is1, a[1, 0], a[0, 0]))
    ah = jnp.where(is2, a[2, 1], jnp.where(is1, a[1, 1], a[0, 1]))
    twx = jnp.log(lw / aw + 1e-16)
    twy = jnp.log(lh / ah + 1e-16)
    clsf = jnp.floor(l0)

    pred = pred_ref[0]
    px = pred[0:1, :]
    py = pred[1:2, :]
    pobj = pred[4:5, :]
    ci = jax.lax.broadcasted_iota(jnp.int32, (1, C), 1)
    ii = (ci % nx).astype(jnp.float32)
    jj = ((ci // nx) % ny).astype(jnp.float32)
    ai = ci // (nx * ny)
    awc = jnp.where(ai == 2, a[2, 0], jnp.where(ai == 1, a[1, 0], a[0, 0]))
    ahc = jnp.where(ai == 2, a[2, 1], jnp.where(ai == 1, a[1, 1], a[0, 1]))
    cx = px + ii
    cy = py + jj
    pwv = jnp.exp(pred[2:3, :]) * awc
    phv = jnp.exp(pred[3:4, :]) * ahc

    wx = (jnp.minimum(cx + 0.5 * pwv, lx + 0.5 * lw)
          - jnp.maximum(cx - 0.5 * pwv, lx - 0.5 * lw))
    wy = (jnp.minimum(cy + 0.5 * phv, ly + 0.5 * lh)
          - jnp.maximum(cy - 0.5 * phv, ly - 0.5 * lh))
    enp = ((wx > 0.0) & (wy > 0.0)).astype(jnp.float32)
    interp = wx * wy * enp
    ioup = interp / (pwv * phv + lw * lh - interp) * validf
    maxiou = jnp.max(ioup, axis=0, keepdims=True)
    ignore = (maxiou > _IGN) & (any_m > 0.0)

    cif = ci.astype(jnp.float32)
    Mf = (cell == cif).astype(jnp.float32) * fin
    pos = jnp.max(Mf, axis=0, keepdims=True)
    scc = jnp.sum(Mf * sc, axis=0, keepdims=True)
    texc = jnp.sum(Mf * tex, axis=0, keepdims=True)
    teyc = jnp.sum(Mf * tey, axis=0, keepdims=True)
    twxc = jnp.sum(Mf * twx, axis=0, keepdims=True)
    twyc = jnp.sum(Mf * twy, axis=0, keepdims=True)
    clsc = jnp.sum(Mf * clsf, axis=0, keepdims=True)

    clamp = lambda z: jnp.maximum(z, -100.0)
    w2 = scc * scc
    pxp = px * pos
    pyp = py * pos
    lxy = (-(texc * clamp(jnp.log(pxp)) + (1.0 - texc) * clamp(jnp.log(1.0 - pxp)))
           - (teyc * clamp(jnp.log(pyp)) + (1.0 - teyc) * clamp(jnp.log(1.0 - pyp)))) * w2
    pws = pred[2:3, :] * pos * scc
    phs = pred[3:4, :] * pos * scc
    lwh = ((pws - twxc * scc) ** 2 + (phs - twyc * scc) ** 2) * 0.5
    objm = jnp.where(pos > 0.0, 1.0, jnp.where(ignore, 0.0, 1.0))
    pop = pobj * objm
    lobj = -(pos * clamp(jnp.log(pop)) + (1.0 - pos) * clamp(jnp.log(1.0 - pop)))
    chi = jax.lax.broadcasted_iota(jnp.int32, (80, 1), 0).astype(jnp.float32)
    T = ((chi == clsc) & (pos > 0.0)).astype(jnp.float32)
    P = pred[5:85, :] * pos
    lcls = -(T * clamp(jnp.log(P)) + (1.0 - T) * clamp(jnp.log(1.0 - P)))

    partial = (jnp.sum(lxy) + jnp.sum(lwh) + jnp.sum(lobj) + jnp.sum(lcls))

    @pl.when(b == 0)
    def _():
        out_ref[...] = jnp.zeros((1, 1), jnp.float32)

    out_ref[...] = out_ref[...] + partial


def _layer_loss(pred, tgt, lyr, size):
    B = pred.shape[0]
    C = 3 * size * size
    pt = jnp.transpose(pred, (0, 4, 1, 2, 3)).reshape(B, 85, C)
    rwh = jnp.asarray((_BASE9 / _STRIDE[lyr]).T)
    out = pl.pallas_call(
        functools.partial(_loss_body, ny=size, nx=size, lyr=lyr),
        grid=(B,),
        in_specs=[
            pl.BlockSpec((1, 85, C), lambda b: (b, 0, 0)),
            pl.BlockSpec((1, _N, 5), lambda b: (b, 0, 0)),
            pl.BlockSpec((2, 9), lambda b: (0, 0)),
        ],
        out_specs=pl.BlockSpec((1, 1), lambda b: (0, 0)),
        out_shape=jax.ShapeDtypeStruct((1, 1), jnp.float32),
    )(pt, tgt, rwh)
    return out[0, 0]


def kernel(pred0, pred1, pred2, targets):
    tgt = targets.astype(jnp.float32)
    l0 = _layer_loss(pred0, tgt, 0, 19)
    l1 = _layer_loss(pred1, tgt, 1, 38)
    l2 = _layer_loss(pred2, tgt, 2, 76)
    return l0 + l1 + l2
